# baseline (speedup 1.0000x reference)
import functools

import jax
import jax.numpy as jnp
from jax import lax
from jax.experimental import pallas as pl
from jax.experimental.pallas import tpu as pltpu
from jax.experimental.pallas import tpu_sc as plsc

N = 10000
E = 320000
D = 128
H = 128
L = 16

NSUB = 16
NTILES = 2 * NSUB
NPAD = 10240
RPT = NPAD // NSUB
CH = 128
NCHUNK = 80
NG = NCHUNK // 4
NCH0 = 160
NCH1 = 0
NQ = 4
CQ = CH // NQ
NROWS2D = NTILES * NCHUNK
XROWS = NROWS2D

_mesh = plsc.VectorSubcoreMesh(core_axis_name="c", subcore_axis_name="s")


@functools.partial(
    pl.kernel,
    mesh=_mesh,
    out_type=jax.ShapeDtypeStruct((2, NPAD, H), jnp.float32),
    scratch_types=[
        pltpu.VMEM((NCHUNK, CH), jnp.int32),
        pltpu.VMEM((CH, H), jnp.float32),
        pltpu.VMEM_SHARED((NPAD, H), jnp.float32),
        pltpu.SemaphoreType.DMA,
        pltpu.SemaphoreType.DMA,
        pltpu.SemaphoreType.DMA,
        pltpu.SemaphoreType.DMA,
    ],
)
def _deg_kernel(dst_hbm, ones_hbm, zeros_hbm, out_hbm,
                didx_v, ones_v, acc_sh, s0, s1, s2, s3):
    ssem = (s0, s1, s2, s3)
    cid = lax.axis_index("c")
    sid = lax.axis_index("s")
    wid = cid * NSUB + sid
    pltpu.sync_copy(dst_hbm.at[pl.ds(wid * NCHUNK, NCHUNK)], didx_v)
    pltpu.sync_copy(zeros_hbm.at[pl.ds(sid * RPT, RPT)],
                    acc_sh.at[pl.ds(sid * RPT, RPT)])
    pltpu.sync_copy(ones_hbm, ones_v)
    plsc.subcore_barrier()

    def group(g, carry):
        for b in range(4):
            pltpu.async_copy(ones_v, acc_sh.at[didx_v.at[g * 4 + b]],
                             ssem[b], add=True)
        for b in range(4):
            pltpu.make_async_copy(
                ones_v, acc_sh.at[didx_v.at[g * 4 + b]], ssem[b]).wait()
        return carry

    lax.fori_loop(0, NG, group, 0)
    plsc.subcore_barrier()
    pltpu.sync_copy(acc_sh.at[pl.ds(sid * RPT, RPT)],
                    out_hbm.at[cid, pl.ds(sid * RPT, RPT)])


@functools.partial(
    pl.kernel,
    mesh=_mesh,
    out_type=jax.ShapeDtypeStruct((2, NPAD, H), jnp.float32),
    scratch_types=[
        pltpu.VMEM((4, CH), jnp.int32),
        pltpu.VMEM((4, CH), jnp.int32),
        pltpu.VMEM((2, CH, H), jnp.float32),
        pltpu.VMEM_SHARED((NPAD, H), jnp.float32),
        pltpu.SemaphoreType.DMA,
        pltpu.SemaphoreType.DMA,
        pltpu.SemaphoreType.DMA,
        pltpu.SemaphoreType.DMA,
        pltpu.SemaphoreType.DMA,
        pltpu.SemaphoreType.DMA,
        pltpu.SemaphoreType.DMA,
        pltpu.SemaphoreType.DMA,
        pltpu.SemaphoreType.DMA,
        pltpu.SemaphoreType.DMA,
        pltpu.SemaphoreType.DMA,
        pltpu.SemaphoreType.DMA,
    ],
)
def _mp_kernel(u_hbm, src_hbm, dst_hbm, zeros_hbm, out_hbm,
               sidx_v, didx_v, rows_v, acc_sh,
               g0, g1, i0, i1, i2, i3, d0, d1, d2, d3, s0, s1):
    gsem = (g0, g1)
    ssem = (s0, s1)
    isem = (i0, i1, i2, i3)
    dsem = (d0, d1, d2, d3)
    cid = lax.axis_index("c")
    sid = lax.axis_index("s")
    nchunk = jnp.where(cid == 0, NCH0, NCH1)
    ng = nchunk // 4
    row0 = jnp.where(cid == 0, sid * NCH0, NSUB * NCH0 + sid * NCH1)
    pltpu.sync_copy(zeros_hbm.at[pl.ds(sid * RPT, RPT)],
                    acc_sh.at[pl.ds(sid * RPT, RPT)])
    plsc.subcore_barrier()

    def start_gather(slot, r):
        for q in range(NQ):
            pltpu.async_copy(
                u_hbm.at[sidx_v.at[slot, pl.ds(q * CQ, CQ)]],
                rows_v.at[r, pl.ds(q * CQ, CQ)], gsem[r])

    def wait_gather(slot, r):
        pltpu.make_async_copy(
            u_hbm.at[sidx_v.at[slot]], rows_v.at[r], gsem[r]).wait()

    @pl.when(cid == 0)
    def _():
        for b in range(4):
            pltpu.async_copy(src_hbm.at[row0 + b], sidx_v.at[b], isem[b])
            pltpu.async_copy(dst_hbm.at[row0 + b], didx_v.at[b], dsem[b])
        pltpu.make_async_copy(src_hbm.at[row0], sidx_v.at[0], isem[0]).wait()
        start_gather(0, 0)

    def group(g, carry):
        for b in range(4):
            j = g * 4 + b
            r = b % 2
            o = 1 - r
            b3 = (b + 3) % 4
            wait_gather(b, r)
            pltpu.make_async_copy(
                dst_hbm.at[row0], didx_v.at[b], dsem[b]).wait()
            pltpu.async_copy(rows_v.at[r], acc_sh.at[didx_v.at[b]],
                             ssem[r], add=True)

            def after_prev_scatter():
                pltpu.make_async_copy(
                    rows_v.at[o], acc_sh.at[didx_v.at[b3]], ssem[o]).wait()

                @pl.when(j + 3 < nchunk)
                def _():
                    pltpu.async_copy(dst_hbm.at[row0 + j + 3],
                                     didx_v.at[b3], dsem[b3])

            if b == 0:
                @pl.when(g > 0)
                def _():
                    after_prev_scatter()
            else:
                after_prev_scatter()

            nb = (b + 1) % 4
            if b < 3:
                pltpu.make_async_copy(
                    src_hbm.at[row0], sidx_v.at[nb], isem[nb]).wait()
                start_gather(nb, o)
            else:
                @pl.when(g < ng - 1)
                def _():
                    pltpu.make_async_copy(
                        src_hbm.at[row0], sidx_v.at[nb], isem[nb]).wait()
                    start_gather(nb, o)
            @pl.when(j + 4 < nchunk)
            def _():
                pltpu.async_copy(src_hbm.at[row0 + j + 4], sidx_v.at[b],
                                 isem[b])
        return carry

    lax.fori_loop(0, ng, group, 0)

    @pl.when(cid == 0)
    def _():
        pltpu.make_async_copy(rows_v.at[1], acc_sh.at[didx_v.at[3]],
                              ssem[1]).wait()
    plsc.subcore_barrier()
    pltpu.sync_copy(acc_sh.at[pl.ds(sid * RPT, RPT)],
                    out_hbm.at[cid, pl.ds(sid * RPT, RPT)])


RB = 400
GRID = N // RB


def _mm1_body(x_ref, w_ref, h_ref):
    h_ref[...] = jnp.dot(x_ref[...], w_ref[...],
                         preferred_element_type=jnp.float32,
                         precision=lax.Precision.HIGHEST)


def _mm1(x, w1):
    return pl.pallas_call(
        _mm1_body,
        grid=(GRID,),
        in_specs=[
            pl.BlockSpec((RB, D), lambda i: (i, 0)),
            pl.BlockSpec((D, H), lambda i: (0, 0)),
        ],
        out_specs=pl.BlockSpec((RB, H), lambda i: (i, 0)),
        out_shape=jax.ShapeDtypeStruct((N, H), jnp.float32),
    )(x, w1)


def _scale1_body(deg_ref, h_ref, dinv_ref, u_ref):
    deg = deg_ref[0, :, 0:1] + deg_ref[1, :, 0:1] + 2.0
    dinv = lax.rsqrt(deg)
    dinv_ref[...] = dinv
    u_ref[...] = h_ref[...] * dinv


def _scale1(deg, h):
    return pl.pallas_call(
        _scale1_body,
        grid=(GRID,),
        in_specs=[
            pl.BlockSpec((2, RB, H), lambda i: (0, i, 0)),
            pl.BlockSpec((RB, H), lambda i: (i, 0)),
        ],
        out_specs=[
            pl.BlockSpec((RB, 1), lambda i: (i, 0)),
            pl.BlockSpec((RB, H), lambda i: (i, 0)),
        ],
        out_shape=[
            jax.ShapeDtypeStruct((N, 1), jnp.float32),
            jax.ShapeDtypeStruct((N, H), jnp.float32),
        ],
    )(deg, h)


def _fin_prep_body(s_ref, u_ref, dinv_ref, b_ref, w_ref, unext_ref):
    dinv = dinv_ref[...]
    s = s_ref[0] + s_ref[1]
    h = jnp.maximum(s * dinv + 2.0 * dinv * u_ref[...] + b_ref[...], 0.0)
    unext_ref[...] = jnp.dot(
        h, w_ref[...], preferred_element_type=jnp.float32,
        precision=lax.Precision.HIGHEST) * dinv


def _fin_prep(s, u, dinv, b, w):
    return pl.pallas_call(
        _fin_prep_body,
        grid=(GRID,),
        in_specs=[
            pl.BlockSpec((2, RB, H), lambda i: (0, i, 0)),
            pl.BlockSpec((RB, H), lambda i: (i, 0)),
            pl.BlockSpec((RB, 1), lambda i: (i, 0)),
            pl.BlockSpec((1, H), lambda i: (0, 0)),
            pl.BlockSpec((H, H), lambda i: (0, 0)),
        ],
        out_specs=pl.BlockSpec((RB, H), lambda i: (i, 0)),
        out_shape=jax.ShapeDtypeStruct((N, H), jnp.float32),
    )(s, u, dinv, b, w)


def _final_body(s_ref, u_ref, dinv_ref, be_ref, wl_ref, bl_ref, out_ref):
    dinv = dinv_ref[...]
    s = s_ref[0] + s_ref[1]
    h = s * dinv + 2.0 * dinv * u_ref[...] + be_ref[...]
    z = jnp.dot(h, wl_ref[...], preferred_element_type=jnp.float32,
                precision=lax.Precision.HIGHEST) + bl_ref[...]
    out_ref[...] = 1.0 / (1.0 + jnp.exp(-z))


def _final(s, u, dinv, be, wl, bl):
    return pl.pallas_call(
        _final_body,
        grid=(GRID,),
        in_specs=[
            pl.BlockSpec((2, RB, H), lambda i: (0, i, 0)),
            pl.BlockSpec((RB, H), lambda i: (i, 0)),
            pl.BlockSpec((RB, 1), lambda i: (i, 0)),
            pl.BlockSpec((1, H), lambda i: (0, 0)),
            pl.BlockSpec((H, L), lambda i: (0, 0)),
            pl.BlockSpec((1, L), lambda i: (0, 0)),
        ],
        out_specs=pl.BlockSpec((RB, L), lambda i: (i, 0)),
        out_shape=jax.ShapeDtypeStruct((N, L), jnp.float32),
    )(s, u, dinv, be, wl, bl)


def kernel(x, edge_index, W1, b1, W2, b2, We, be, Wl, bl):
    ei = edge_index.astype(jnp.int32)
    pad = XROWS * CH - E
    src_p = jnp.concatenate([ei[0], jnp.zeros((pad,), jnp.int32)])
    dst_p = jnp.concatenate([ei[1], jnp.full((pad,), N, jnp.int32)])
    src2d = src_p.reshape(XROWS, CH)
    dst2d = dst_p.reshape(XROWS, CH)

    ones1 = jnp.ones((CH, H), jnp.float32)
    z2 = jnp.zeros((NPAD, H), jnp.float32)

    deg = _deg_kernel(dst2d, ones1, z2)
    h1 = _mm1(x, W1)
    dinv, u1 = _scale1(deg, h1)
    s1 = _mp_kernel(u1, src2d, dst2d, z2)
    u2 = _fin_prep(s1, u1, dinv, b1.reshape(1, H), W2)
    s2 = _mp_kernel(u2, src2d, dst2d, z2)
    u3 = _fin_prep(s2, u2, dinv, b2.reshape(1, H), We)
    s3 = _mp_kernel(u3, src2d, dst2d, z2)
    return _final(s3, u3, dinv, be.reshape(1, H), Wl, bl.reshape(1, L))


# speedup vs baseline: 1.3609x; 1.3609x over previous
import functools

import jax
import jax.numpy as jnp
from jax import lax
from jax.experimental import pallas as pl
from jax.experimental.pallas import tpu as pltpu
from jax.experimental.pallas import tpu_sc as plsc

N = 10000
E = 320000
D = 128
H = 128
L = 16

NSUB = 16
NTILES = 2 * NSUB
NPAD = 10240
RPT = NPAD // NSUB
CH = 128
NCHUNK = 80
NG = NCHUNK // 4
NCH0 = 136
NCH1 = 24
NQ = 4
CQ = CH // NQ
NROWS2D = NTILES * NCHUNK
XROWS = NROWS2D

_mesh = plsc.VectorSubcoreMesh(core_axis_name="c", subcore_axis_name="s")


@functools.partial(
    pl.kernel,
    mesh=_mesh,
    out_type=jax.ShapeDtypeStruct((2, NPAD, H), jnp.float32),
    scratch_types=[
        pltpu.VMEM((NCHUNK, CH), jnp.int32),
        pltpu.VMEM((CH, H), jnp.float32),
        pltpu.VMEM_SHARED((NPAD, H), jnp.float32),
        pltpu.SemaphoreType.DMA,
        pltpu.SemaphoreType.DMA,
        pltpu.SemaphoreType.DMA,
        pltpu.SemaphoreType.DMA,
    ],
)
def _deg_kernel(dst_hbm, ones_hbm, zeros_hbm, out_hbm,
                didx_v, ones_v, acc_sh, s0, s1, s2, s3):
    ssem = (s0, s1, s2, s3)
    cid = lax.axis_index("c")
    sid = lax.axis_index("s")
    wid = cid * NSUB + sid
    pltpu.sync_copy(dst_hbm.at[pl.ds(wid * NCHUNK, NCHUNK)], didx_v)
    pltpu.sync_copy(zeros_hbm.at[pl.ds(sid * RPT, RPT)],
                    acc_sh.at[pl.ds(sid * RPT, RPT)])
    pltpu.sync_copy(ones_hbm, ones_v)
    plsc.subcore_barrier()

    def group(g, carry):
        for b in range(4):
            pltpu.async_copy(ones_v, acc_sh.at[didx_v.at[g * 4 + b]],
                             ssem[b], add=True)
        for b in range(4):
            pltpu.make_async_copy(
                ones_v, acc_sh.at[didx_v.at[g * 4 + b]], ssem[b]).wait()
        return carry

    lax.fori_loop(0, NG, group, 0)
    plsc.subcore_barrier()
    pltpu.sync_copy(acc_sh.at[pl.ds(sid * RPT, RPT)],
                    out_hbm.at[cid, pl.ds(sid * RPT, RPT)])


@functools.partial(
    pl.kernel,
    mesh=_mesh,
    out_type=jax.ShapeDtypeStruct((2, NPAD, H), jnp.float32),
    scratch_types=[
        pltpu.VMEM((4, CH), jnp.int32),
        pltpu.VMEM((4, CH), jnp.int32),
        pltpu.VMEM((2, CH, H), jnp.float32),
        pltpu.VMEM_SHARED((NPAD, H), jnp.float32),
        pltpu.SemaphoreType.DMA,
        pltpu.SemaphoreType.DMA,
        pltpu.SemaphoreType.DMA,
        pltpu.SemaphoreType.DMA,
        pltpu.SemaphoreType.DMA,
        pltpu.SemaphoreType.DMA,
        pltpu.SemaphoreType.DMA,
        pltpu.SemaphoreType.DMA,
        pltpu.SemaphoreType.DMA,
        pltpu.SemaphoreType.DMA,
        pltpu.SemaphoreType.DMA,
        pltpu.SemaphoreType.DMA,
    ],
)
def _mp_kernel(u_hbm, src_hbm, dst_hbm, zeros_hbm, out_hbm,
               sidx_v, didx_v, rows_v, acc_sh,
               g0, g1, i0, i1, i2, i3, d0, d1, d2, d3, s0, s1):
    gsem = (g0, g1)
    ssem = (s0, s1)
    isem = (i0, i1, i2, i3)
    dsem = (d0, d1, d2, d3)
    cid = lax.axis_index("c")
    sid = lax.axis_index("s")
    nchunk = jnp.where(cid == 0, NCH0, NCH1)
    ng = nchunk // 4
    row0 = jnp.where(cid == 0, sid * NCH0, NSUB * NCH0 + sid * NCH1)
    pltpu.sync_copy(zeros_hbm.at[pl.ds(sid * RPT, RPT)],
                    acc_sh.at[pl.ds(sid * RPT, RPT)])
    plsc.subcore_barrier()

    def start_gather(slot, r):
        for q in range(NQ):
            pltpu.async_copy(
                u_hbm.at[sidx_v.at[slot, pl.ds(q * CQ, CQ)]],
                rows_v.at[r, pl.ds(q * CQ, CQ)], gsem[r])

    def wait_gather(slot, r):
        pltpu.make_async_copy(
            u_hbm.at[sidx_v.at[slot]], rows_v.at[r], gsem[r]).wait()

    for b in range(4):
        pltpu.async_copy(src_hbm.at[row0 + b], sidx_v.at[b], isem[b])
        pltpu.async_copy(dst_hbm.at[row0 + b], didx_v.at[b], dsem[b])
    pltpu.make_async_copy(src_hbm.at[row0], sidx_v.at[0], isem[0]).wait()
    start_gather(0, 0)

    def group(g, carry):
        for b in range(4):
            j = g * 4 + b
            r = b % 2
            o = 1 - r
            b3 = (b + 3) % 4
            wait_gather(b, r)
            pltpu.make_async_copy(
                dst_hbm.at[row0], didx_v.at[b], dsem[b]).wait()
            pltpu.async_copy(rows_v.at[r], acc_sh.at[didx_v.at[b]],
                             ssem[r], add=True)

            def after_prev_scatter():
                pltpu.make_async_copy(
                    rows_v.at[o], acc_sh.at[didx_v.at[b3]], ssem[o]).wait()

                @pl.when(j + 3 < nchunk)
                def _():
                    pltpu.async_copy(dst_hbm.at[row0 + j + 3],
                                     didx_v.at[b3], dsem[b3])

            if b == 0:
                @pl.when(g > 0)
                def _():
                    after_prev_scatter()
            else:
                after_prev_scatter()

            nb = (b + 1) % 4
            if b < 3:
                pltpu.make_async_copy(
                    src_hbm.at[row0], sidx_v.at[nb], isem[nb]).wait()
                start_gather(nb, o)
            else:
                @pl.when(g < ng - 1)
                def _():
                    pltpu.make_async_copy(
                        src_hbm.at[row0], sidx_v.at[nb], isem[nb]).wait()
                    start_gather(nb, o)
            @pl.when(j + 4 < nchunk)
            def _():
                pltpu.async_copy(src_hbm.at[row0 + j + 4], sidx_v.at[b],
                                 isem[b])
        return carry

    lax.fori_loop(0, ng, group, 0)
    pltpu.make_async_copy(rows_v.at[1], acc_sh.at[didx_v.at[3]],
                          ssem[1]).wait()
    plsc.subcore_barrier()
    pltpu.sync_copy(acc_sh.at[pl.ds(sid * RPT, RPT)],
                    out_hbm.at[cid, pl.ds(sid * RPT, RPT)])


RB = 400
GRID = N // RB


def _mm1_body(x_ref, w_ref, h_ref):
    h_ref[...] = jnp.dot(x_ref[...], w_ref[...],
                         preferred_element_type=jnp.float32,
                         precision=lax.Precision.HIGHEST)


def _mm1(x, w1):
    return pl.pallas_call(
        _mm1_body,
        grid=(GRID,),
        in_specs=[
            pl.BlockSpec((RB, D), lambda i: (i, 0)),
            pl.BlockSpec((D, H), lambda i: (0, 0)),
        ],
        out_specs=pl.BlockSpec((RB, H), lambda i: (i, 0)),
        out_shape=jax.ShapeDtypeStruct((N, H), jnp.float32),
    )(x, w1)


def _scale1_body(deg_ref, h_ref, dinv_ref, u_ref):
    deg = deg_ref[0, :, 0:1] + deg_ref[1, :, 0:1] + 2.0
    dinv = lax.rsqrt(deg)
    dinv_ref[...] = dinv
    u_ref[...] = h_ref[...] * dinv


def _scale1(deg, h):
    return pl.pallas_call(
        _scale1_body,
        grid=(GRID,),
        in_specs=[
            pl.BlockSpec((2, RB, H), lambda i: (0, i, 0)),
            pl.BlockSpec((RB, H), lambda i: (i, 0)),
        ],
        out_specs=[
            pl.BlockSpec((RB, 1), lambda i: (i, 0)),
            pl.BlockSpec((RB, H), lambda i: (i, 0)),
        ],
        out_shape=[
            jax.ShapeDtypeStruct((N, 1), jnp.float32),
            jax.ShapeDtypeStruct((N, H), jnp.float32),
        ],
    )(deg, h)


def _fin_prep_body(s_ref, u_ref, dinv_ref, b_ref, w_ref, unext_ref):
    dinv = dinv_ref[...]
    s = s_ref[0] + s_ref[1]
    h = jnp.maximum(s * dinv + 2.0 * dinv * u_ref[...] + b_ref[...], 0.0)
    unext_ref[...] = jnp.dot(
        h, w_ref[...], preferred_element_type=jnp.float32,
        precision=lax.Precision.HIGHEST) * dinv


def _fin_prep(s, u, dinv, b, w):
    return pl.pallas_call(
        _fin_prep_body,
        grid=(GRID,),
        in_specs=[
            pl.BlockSpec((2, RB, H), lambda i: (0, i, 0)),
            pl.BlockSpec((RB, H), lambda i: (i, 0)),
            pl.BlockSpec((RB, 1), lambda i: (i, 0)),
            pl.BlockSpec((1, H), lambda i: (0, 0)),
            pl.BlockSpec((H, H), lambda i: (0, 0)),
        ],
        out_specs=pl.BlockSpec((RB, H), lambda i: (i, 0)),
        out_shape=jax.ShapeDtypeStruct((N, H), jnp.float32),
    )(s, u, dinv, b, w)


def _final_body(s_ref, u_ref, dinv_ref, be_ref, wl_ref, bl_ref, out_ref):
    dinv = dinv_ref[...]
    s = s_ref[0] + s_ref[1]
    h = s * dinv + 2.0 * dinv * u_ref[...] + be_ref[...]
    z = jnp.dot(h, wl_ref[...], preferred_element_type=jnp.float32,
                precision=lax.Precision.HIGHEST) + bl_ref[...]
    out_ref[...] = 1.0 / (1.0 + jnp.exp(-z))


def _final(s, u, dinv, be, wl, bl):
    return pl.pallas_call(
        _final_body,
        grid=(GRID,),
        in_specs=[
            pl.BlockSpec((2, RB, H), lambda i: (0, i, 0)),
            pl.BlockSpec((RB, H), lambda i: (i, 0)),
            pl.BlockSpec((RB, 1), lambda i: (i, 0)),
            pl.BlockSpec((1, H), lambda i: (0, 0)),
            pl.BlockSpec((H, L), lambda i: (0, 0)),
            pl.BlockSpec((1, L), lambda i: (0, 0)),
        ],
        out_specs=pl.BlockSpec((RB, L), lambda i: (i, 0)),
        out_shape=jax.ShapeDtypeStruct((N, L), jnp.float32),
    )(s, u, dinv, be, wl, bl)


def kernel(x, edge_index, W1, b1, W2, b2, We, be, Wl, bl):
    ei = edge_index.astype(jnp.int32)
    pad = XROWS * CH - E
    src_p = jnp.concatenate([ei[0], jnp.zeros((pad,), jnp.int32)])
    dst_p = jnp.concatenate([ei[1], jnp.full((pad,), N, jnp.int32)])
    src2d = src_p.reshape(XROWS, CH)
    dst2d = dst_p.reshape(XROWS, CH)

    ones1 = jnp.ones((CH, H), jnp.float32)
    z2 = jnp.zeros((NPAD, H), jnp.float32)

    deg = _deg_kernel(dst2d, ones1, z2)
    h1 = _mm1(x, W1)
    dinv, u1 = _scale1(deg, h1)
    s1 = _mp_kernel(u1, src2d, dst2d, z2)
    u2 = _fin_prep(s1, u1, dinv, b1.reshape(1, H), W2)
    s2 = _mp_kernel(u2, src2d, dst2d, z2)
    u3 = _fin_prep(s2, u2, dinv, b2.reshape(1, H), We)
    s3 = _mp_kernel(u3, src2d, dst2d, z2)
    return _final(s3, u3, dinv, be.reshape(1, H), Wl, bl.reshape(1, L))
